# baseline (speedup 1.0000x reference)
import jax
import jax.numpy as jnp
from jax.experimental import pallas as pl
from jax.experimental.pallas import tpu as pltpu

N = 8192
BLK = 32
W3 = 0.25


def _hd_body(adv_t_ref, ori_ref, out_ref, m_ref):
    a0 = adv_t_ref[0:1, :]
    a1 = adv_t_ref[1:2, :]
    a2 = adv_t_ref[2:3, :]
    a3 = adv_t_ref[3:4, :]

    m_ref[...] = jnp.full((BLK, N), jnp.inf, jnp.float32)

    def body(j, _):
        o = ori_ref[pl.ds(j * BLK, BLK), :]
        d0 = a0 - o[:, 0:1]
        d1 = a1 - o[:, 1:2]
        d2 = a2 - o[:, 2:3]
        d3 = a3 - o[:, 3:4]
        acc = d0 * d0 + d1 * d1 + d2 * d2 + W3 * (d3 * d3)
        m_ref[...] = jnp.minimum(m_ref[...], acc)
        return 0

    jax.lax.fori_loop(0, N // BLK, body, 0)

    nn = jnp.min(m_ref[...], axis=0)
    out_ref[...] = jnp.max(nn).reshape(1, 1)


def kernel(adv_pc, ori_pc):
    adv_t = adv_pc.T
    out = pl.pallas_call(
        _hd_body,
        out_shape=jax.ShapeDtypeStruct((1, 1), jnp.float32),
        scratch_shapes=[pltpu.VMEM((BLK, N), jnp.float32)],
    )(adv_t, ori_pc)
    return out.reshape(1)


# speedup vs baseline: 1.2472x; 1.2472x over previous
import jax
import jax.numpy as jnp
from jax.experimental import pallas as pl
from jax.experimental.pallas import tpu as pltpu

N = 8192
BLK = 256
W3 = 0.25


def _hd_body(adv_t_ref, ori_ref, out_ref, aaug_ref, baug_ref, m_ref):
    a0 = adv_t_ref[0:1, :]
    a1 = adv_t_ref[1:2, :]
    a2 = adv_t_ref[2:3, :]
    a3 = adv_t_ref[3:4, :]
    na = a0 * a0 + a1 * a1 + a2 * a2 + W3 * (a3 * a3)
    ones_r = jnp.ones((1, N), jnp.float32)
    aaug_ref[...] = jnp.concatenate(
        [a0, a1, a2, a3, ones_r, na, jnp.zeros((2, N), jnp.float32)], axis=0
    )

    b0 = ori_ref[:, 0:1]
    b1 = ori_ref[:, 1:2]
    b2 = ori_ref[:, 2:3]
    b3 = ori_ref[:, 3:4]
    nb = b0 * b0 + b1 * b1 + b2 * b2 + W3 * (b3 * b3)
    ones_c = jnp.ones((N, 1), jnp.float32)
    baug_ref[...] = jnp.concatenate(
        [
            -2.0 * b0,
            -2.0 * b1,
            -2.0 * b2,
            (-2.0 * W3) * b3,
            nb,
            ones_c,
            jnp.zeros((N, 2), jnp.float32),
        ],
        axis=1,
    )

    m_ref[...] = jnp.full((BLK, N), jnp.inf, jnp.float32)

    def body(j, _):
        b_blk = baug_ref[pl.ds(j * BLK, BLK), :]
        e = jax.lax.dot_general(
            b_blk,
            aaug_ref[...],
            (((1,), (0,)), ((), ())),
            preferred_element_type=jnp.float32,
            precision=jax.lax.Precision.HIGHEST,
        )
        m_ref[...] = jnp.minimum(m_ref[...], e)
        return 0

    jax.lax.fori_loop(0, N // BLK, body, 0)

    nn = jnp.min(m_ref[...], axis=0)
    out_ref[...] = jnp.max(nn).reshape(1, 1)


def kernel(adv_pc, ori_pc):
    adv_t = adv_pc.T
    out = pl.pallas_call(
        _hd_body,
        out_shape=jax.ShapeDtypeStruct((1, 1), jnp.float32),
        scratch_shapes=[
            pltpu.VMEM((8, N), jnp.float32),
            pltpu.VMEM((N, 8), jnp.float32),
            pltpu.VMEM((BLK, N), jnp.float32),
        ],
    )(adv_t, ori_pc)
    return out.reshape(1)


# speedup vs baseline: 4.5364x; 3.6373x over previous
import jax
import jax.numpy as jnp
from jax.experimental import pallas as pl
from jax.experimental.pallas import tpu as pltpu

N = 8192
BLK = 256
K = 24
W3 = 0.25


def _split(x):
    hi = x.astype(jnp.bfloat16).astype(jnp.float32)
    lo = (x - hi).astype(jnp.bfloat16).astype(jnp.float32)
    return hi, lo


def _hd_body(adv_t_ref, ori_ref, out_ref, aaug_ref, baug_ref, m_ref):
    a0 = adv_t_ref[0:1, :]
    a1 = adv_t_ref[1:2, :]
    a2 = adv_t_ref[2:3, :]
    a3 = adv_t_ref[3:4, :]
    na = a0 * a0 + a1 * a1 + a2 * a2 + W3 * (a3 * a3)
    ah0, al0 = _split(a0)
    ah1, al1 = _split(a1)
    ah2, al2 = _split(a2)
    ah3, al3 = _split(a3)
    nah, nal = _split(na)
    ones_r = jnp.ones((1, N), jnp.float32)
    aaug_ref[...] = jnp.concatenate(
        [
            ah0, ah1, ah2, ah3,
            ah0, ah1, ah2, ah3,
            al0, al1, al2, al3,
            al0, al1, al2, al3,
            ones_r, ones_r, nah, nal,
            jnp.zeros((K - 20, N), jnp.float32),
        ],
        axis=0,
    )

    b0 = ori_ref[:, 0:1]
    b1 = ori_ref[:, 1:2]
    b2 = ori_ref[:, 2:3]
    b3 = ori_ref[:, 3:4]
    nb = b0 * b0 + b1 * b1 + b2 * b2 + W3 * (b3 * b3)
    bh0, bl0 = _split(b0)
    bh1, bl1 = _split(b1)
    bh2, bl2 = _split(b2)
    bh3, bl3 = _split(b3)
    nbh, nbl = _split(nb)
    ones_c = jnp.ones((N, 1), jnp.float32)
    baug_ref[...] = jnp.concatenate(
        [
            -2.0 * bh0, -2.0 * bh1, -2.0 * bh2, -0.5 * bh3,
            -2.0 * bl0, -2.0 * bl1, -2.0 * bl2, -0.5 * bl3,
            -2.0 * bh0, -2.0 * bh1, -2.0 * bh2, -0.5 * bh3,
            -2.0 * bl0, -2.0 * bl1, -2.0 * bl2, -0.5 * bl3,
            nbh, nbl, ones_c, ones_c,
            jnp.zeros((N, K - 20), jnp.float32),
        ],
        axis=1,
    )

    m_ref[...] = jnp.full((BLK, N), jnp.inf, jnp.float32)

    def body(j, _):
        b_blk = baug_ref[pl.ds(j * BLK, BLK), :]
        e = jax.lax.dot_general(
            b_blk,
            aaug_ref[...],
            (((1,), (0,)), ((), ())),
            preferred_element_type=jnp.float32,
        )
        m_ref[...] = jnp.minimum(m_ref[...], e)
        return 0

    jax.lax.fori_loop(0, N // BLK, body, 0)

    nn = jnp.min(m_ref[...], axis=0)
    out_ref[...] = jnp.max(nn).reshape(1, 1)


def kernel(adv_pc, ori_pc):
    adv_t = adv_pc.T
    out = pl.pallas_call(
        _hd_body,
        out_shape=jax.ShapeDtypeStruct((1, 1), jnp.float32),
        scratch_shapes=[
            pltpu.VMEM((K, N), jnp.float32),
            pltpu.VMEM((N, K), jnp.float32),
            pltpu.VMEM((BLK, N), jnp.float32),
        ],
    )(adv_t, ori_pc)
    return out.reshape(1)


# speedup vs baseline: 4.9518x; 1.0916x over previous
import jax
import jax.numpy as jnp
from jax.experimental import pallas as pl
from jax.experimental.pallas import tpu as pltpu

N = 8192
BLK = 256
K = 24
W3 = 0.25


def _split(x):
    hi = x.astype(jnp.bfloat16)
    lo = (x - hi.astype(jnp.float32)).astype(jnp.bfloat16)
    return hi, lo


def _hd_body(adv_t_ref, ori_ref, out_ref, aaug_ref, baug_ref):
    a0 = adv_t_ref[0:1, :]
    a1 = adv_t_ref[1:2, :]
    a2 = adv_t_ref[2:3, :]
    a3 = adv_t_ref[3:4, :]
    na = a0 * a0 + a1 * a1 + a2 * a2 + W3 * (a3 * a3)
    ah0, al0 = _split(a0)
    ah1, al1 = _split(a1)
    ah2, al2 = _split(a2)
    ah3, al3 = _split(a3)
    nah, nal = _split(na)
    ones_r = jnp.ones((1, N), jnp.bfloat16)
    aaug_ref[...] = jnp.concatenate(
        [
            ah0, ah1, ah2, ah3,
            ah0, ah1, ah2, ah3,
            al0, al1, al2, al3,
            al0, al1, al2, al3,
            ones_r, ones_r, nah, nal,
            jnp.zeros((K - 20, N), jnp.bfloat16),
        ],
        axis=0,
    )

    b0 = ori_ref[:, 0:1]
    b1 = ori_ref[:, 1:2]
    b2 = ori_ref[:, 2:3]
    b3 = ori_ref[:, 3:4]
    nb = b0 * b0 + b1 * b1 + b2 * b2 + W3 * (b3 * b3)
    bh0, bl0 = _split(-2.0 * b0)
    bh1, bl1 = _split(-2.0 * b1)
    bh2, bl2 = _split(-2.0 * b2)
    bh3, bl3 = _split(-0.5 * b3)
    nbh, nbl = _split(nb)
    ones_c = jnp.ones((N, 1), jnp.bfloat16)
    baug_ref[...] = jnp.concatenate(
        [
            bh0, bh1, bh2, bh3,
            bl0, bl1, bl2, bl3,
            bh0, bh1, bh2, bh3,
            bl0, bl1, bl2, bl3,
            nbh, nbl, ones_c, ones_c,
            jnp.zeros((N, K - 20), jnp.bfloat16),
        ],
        axis=1,
    )

    def body(j, m):
        b_blk = baug_ref[pl.ds(j * BLK, BLK), :]
        e = jax.lax.dot_general(
            b_blk,
            aaug_ref[...],
            (((1,), (0,)), ((), ())),
            preferred_element_type=jnp.float32,
        )
        e8 = jnp.min(e.reshape(BLK // 8, 8, N), axis=0)
        return jnp.minimum(m, e8)

    m = jax.lax.fori_loop(
        0, N // BLK, body, jnp.full((8, N), jnp.inf, jnp.float32)
    )

    nn = jnp.min(m, axis=0)
    out_ref[...] = jnp.max(nn).reshape(1, 1)


def kernel(adv_pc, ori_pc):
    adv_t = adv_pc.T
    out = pl.pallas_call(
        _hd_body,
        out_shape=jax.ShapeDtypeStruct((1, 1), jnp.float32),
        scratch_shapes=[
            pltpu.VMEM((K, N), jnp.bfloat16),
            pltpu.VMEM((N, K), jnp.bfloat16),
        ],
    )(adv_t, ori_pc)
    return out.reshape(1)


# speedup vs baseline: 6.2071x; 1.2535x over previous
import jax
import jax.numpy as jnp
from jax.experimental import pallas as pl
from jax.experimental.pallas import tpu as pltpu

N = 8192
BLK = 256
K = 24
W3 = 0.25


def _split(x):
    hi = x.astype(jnp.bfloat16)
    lo = (x - hi.astype(jnp.float32)).astype(jnp.bfloat16)
    return hi, lo


def _hd_body(adv_t_ref, ori_t_ref, out_ref, aaug_ref, baug_ref):
    a0 = adv_t_ref[0:1, :]
    a1 = adv_t_ref[1:2, :]
    a2 = adv_t_ref[2:3, :]
    a3 = adv_t_ref[3:4, :]
    na = a0 * a0 + a1 * a1 + a2 * a2 + W3 * (a3 * a3)
    ah0, al0 = _split(a0)
    ah1, al1 = _split(a1)
    ah2, al2 = _split(a2)
    ah3, al3 = _split(a3)
    nah, nal = _split(na)
    ones_r = jnp.ones((1, N), jnp.bfloat16)
    aaug_ref[...] = jnp.concatenate(
        [
            ah0, ah1, ah2, ah3,
            ah0, ah1, ah2, ah3,
            al0, al1, al2, al3,
            al0, al1, al2, al3,
            ones_r, ones_r, nah, nal,
            jnp.zeros((K - 20, N), jnp.bfloat16),
        ],
        axis=0,
    )

    b0 = ori_t_ref[0:1, :]
    b1 = ori_t_ref[1:2, :]
    b2 = ori_t_ref[2:3, :]
    b3 = ori_t_ref[3:4, :]
    nb = b0 * b0 + b1 * b1 + b2 * b2 + W3 * (b3 * b3)
    bh0, bl0 = _split(-2.0 * b0)
    bh1, bl1 = _split(-2.0 * b1)
    bh2, bl2 = _split(-2.0 * b2)
    bh3, bl3 = _split(-0.5 * b3)
    nbh, nbl = _split(nb)
    baug_ref[...] = jnp.concatenate(
        [
            bh0, bh1, bh2, bh3,
            bl0, bl1, bl2, bl3,
            bh0, bh1, bh2, bh3,
            bl0, bl1, bl2, bl3,
            nbh, nbl, ones_r, ones_r,
            jnp.zeros((K - 20, N), jnp.bfloat16),
        ],
        axis=0,
    )

    def body(j, m):
        b_blk = baug_ref[:, pl.ds(j * BLK, BLK)]
        e = jax.lax.dot_general(
            b_blk,
            aaug_ref[...],
            (((0,), (0,)), ((), ())),
            preferred_element_type=jnp.float32,
        )
        e8 = jnp.min(e.reshape(BLK // 8, 8, N), axis=0)
        return jnp.minimum(m, e8)

    m = jax.lax.fori_loop(
        0, N // BLK, body, jnp.full((8, N), jnp.inf, jnp.float32)
    )

    nn = jnp.min(m, axis=0)
    out_ref[...] = jnp.max(nn).reshape(1, 1)


def kernel(adv_pc, ori_pc):
    adv_t = adv_pc.T
    ori_t = ori_pc.T
    out = pl.pallas_call(
        _hd_body,
        out_shape=jax.ShapeDtypeStruct((1, 1), jnp.float32),
        scratch_shapes=[
            pltpu.VMEM((K, N), jnp.bfloat16),
            pltpu.VMEM((K, N), jnp.bfloat16),
        ],
    )(adv_t, ori_t)
    return out.reshape(1)


# speedup vs baseline: 6.7545x; 1.0882x over previous
import jax
import jax.numpy as jnp
from jax.experimental import pallas as pl
from jax.experimental.pallas import tpu as pltpu

N = 8192
BLK = 256
K = 24
W3 = 0.25


def _split(x):
    hi = x.astype(jnp.bfloat16)
    lo = (x - hi.astype(jnp.float32)).astype(jnp.bfloat16)
    return hi, lo


def _hd_body(adv_t_ref, ori_t_ref, out_ref, aaug_ref, baug_ref):
    a0 = adv_t_ref[0:1, :]
    a1 = adv_t_ref[1:2, :]
    a2 = adv_t_ref[2:3, :]
    a3 = adv_t_ref[3:4, :]
    na = a0 * a0 + a1 * a1 + a2 * a2 + W3 * (a3 * a3)
    ah0, al0 = _split(a0)
    ah1, al1 = _split(a1)
    ah2, al2 = _split(a2)
    ah3, al3 = _split(a3)
    nah, nal = _split(na)
    ones_r = jnp.ones((1, N), jnp.bfloat16)
    aaug_ref[...] = jnp.concatenate(
        [
            ah0, ah1, ah2, ah3,
            ah0, ah1, ah2, ah3,
            al0, al1, al2, al3,
            al0, al1, al2, al3,
            ones_r, ones_r, nah, nal,
            jnp.zeros((K - 20, N), jnp.bfloat16),
        ],
        axis=0,
    )

    b0 = ori_t_ref[0:1, :]
    b1 = ori_t_ref[1:2, :]
    b2 = ori_t_ref[2:3, :]
    b3 = ori_t_ref[3:4, :]
    nb = b0 * b0 + b1 * b1 + b2 * b2 + W3 * (b3 * b3)
    bh0, bl0 = _split(-2.0 * b0)
    bh1, bl1 = _split(-2.0 * b1)
    bh2, bl2 = _split(-2.0 * b2)
    bh3, bl3 = _split(-0.5 * b3)
    nbh, nbl = _split(nb)
    baug_ref[...] = jnp.concatenate(
        [
            bh0, bh1, bh2, bh3,
            bl0, bl1, bl2, bl3,
            bh0, bh1, bh2, bh3,
            bl0, bl1, bl2, bl3,
            nbh, nbl, ones_r, ones_r,
            jnp.zeros((K - 20, N), jnp.bfloat16),
        ],
        axis=0,
    )

    def _tile_min(j):
        b_blk = baug_ref[:, pl.ds(j * BLK, BLK)]
        e = jax.lax.dot_general(
            b_blk,
            aaug_ref[...],
            (((0,), (0,)), ((), ())),
            preferred_element_type=jnp.float32,
        )
        return jnp.min(e.reshape(BLK // 8, 8, N), axis=0)

    def body(jj, m):
        e8a = _tile_min(2 * jj)
        e8b = _tile_min(2 * jj + 1)
        return jnp.minimum(m, jnp.minimum(e8a, e8b))

    m = jax.lax.fori_loop(
        0, N // (2 * BLK), body, jnp.full((8, N), jnp.inf, jnp.float32)
    )

    nn = jnp.min(m, axis=0)
    out_ref[...] = jnp.max(nn).reshape(1, 1)


def kernel(adv_pc, ori_pc):
    adv_t = adv_pc.T
    ori_t = ori_pc.T
    out = pl.pallas_call(
        _hd_body,
        out_shape=jax.ShapeDtypeStruct((1, 1), jnp.float32),
        scratch_shapes=[
            pltpu.VMEM((K, N), jnp.bfloat16),
            pltpu.VMEM((K, N), jnp.bfloat16),
        ],
    )(adv_t, ori_t)
    return out.reshape(1)


# speedup vs baseline: 7.0169x; 1.0389x over previous
import jax
import jax.numpy as jnp
from jax.experimental import pallas as pl
from jax.experimental.pallas import tpu as pltpu

N = 8192
BLK = 256
K = 24
W3 = 0.25


def _split(x):
    hi = x.astype(jnp.bfloat16)
    lo = (x - hi.astype(jnp.float32)).astype(jnp.bfloat16)
    return hi, lo


def _hd_body(adv_t_ref, ori_t_ref, out_ref, aaug_ref, baug_ref):
    a0 = adv_t_ref[0:1, :]
    a1 = adv_t_ref[1:2, :]
    a2 = adv_t_ref[2:3, :]
    a3 = adv_t_ref[3:4, :]
    na = a0 * a0 + a1 * a1 + a2 * a2 + W3 * (a3 * a3)
    ah0, al0 = _split(a0)
    ah1, al1 = _split(a1)
    ah2, al2 = _split(a2)
    ah3, al3 = _split(a3)
    nah, nal = _split(na)
    ones_r = jnp.ones((1, N), jnp.bfloat16)
    aaug_ref[...] = jnp.concatenate(
        [
            ah0, ah1, ah2, ah3,
            ah0, ah1, ah2, ah3,
            al0, al1, al2, al3,
            al0, al1, al2, al3,
            ones_r, ones_r, nah, nal,
            jnp.zeros((K - 20, N), jnp.bfloat16),
        ],
        axis=0,
    )

    b0 = ori_t_ref[0:1, :]
    b1 = ori_t_ref[1:2, :]
    b2 = ori_t_ref[2:3, :]
    b3 = ori_t_ref[3:4, :]
    nb = b0 * b0 + b1 * b1 + b2 * b2 + W3 * (b3 * b3)
    bh0, bl0 = _split(-2.0 * b0)
    bh1, bl1 = _split(-2.0 * b1)
    bh2, bl2 = _split(-2.0 * b2)
    bh3, bl3 = _split(-0.5 * b3)
    nbh, nbl = _split(nb)
    baug_ref[...] = jnp.concatenate(
        [
            bh0, bh1, bh2, bh3,
            bl0, bl1, bl2, bl3,
            bh0, bh1, bh2, bh3,
            bl0, bl1, bl2, bl3,
            nbh, nbl, ones_r, ones_r,
            jnp.zeros((K - 20, N), jnp.bfloat16),
        ],
        axis=0,
    )

    def _tile_min(j):
        b_blk = baug_ref[:, pl.ds(j * BLK, BLK)]
        e = jax.lax.dot_general(
            b_blk,
            aaug_ref[...],
            (((0,), (0,)), ((), ())),
            preferred_element_type=jnp.float32,
        )
        return jnp.min(e.reshape(BLK // 8, 8, N), axis=0)

    def body(jj, m):
        e8a = jnp.minimum(_tile_min(4 * jj), _tile_min(4 * jj + 1))
        e8b = jnp.minimum(_tile_min(4 * jj + 2), _tile_min(4 * jj + 3))
        return jnp.minimum(m, jnp.minimum(e8a, e8b))

    m = jax.lax.fori_loop(
        0, N // (4 * BLK), body, jnp.full((8, N), jnp.inf, jnp.float32)
    )

    nn = jnp.min(m, axis=0)
    out_ref[...] = jnp.max(nn).reshape(1, 1)


def kernel(adv_pc, ori_pc):
    adv_t = adv_pc.T
    ori_t = ori_pc.T
    out = pl.pallas_call(
        _hd_body,
        out_shape=jax.ShapeDtypeStruct((1, 1), jnp.float32),
        scratch_shapes=[
            pltpu.VMEM((K, N), jnp.bfloat16),
            pltpu.VMEM((K, N), jnp.bfloat16),
        ],
    )(adv_t, ori_t)
    return out.reshape(1)
